# Initial kernel scaffold; baseline (speedup 1.0000x reference)
import functools

import jax
import jax.numpy as jnp
from jax import lax
from jax.experimental import pallas as pl
from jax.experimental.pallas import tpu as pltpu
from jax.experimental.pallas import tpu_sc as plsc

_LANES = 16


def _demand_body(rows_per_w, num_cores, batch_hbm, a_hbm, b_hbm, out_hbm,
                 idx_v, a_v, b_v, out_v):
    c = lax.axis_index("c")
    s = lax.axis_index("s")
    wid = s * num_cores + c
    base = wid * rows_per_w

    pltpu.sync_copy(a_hbm, a_v)
    pltpu.sync_copy(b_hbm, b_v)
    pltpu.sync_copy(batch_hbm.at[pl.ds(base * 2, rows_per_w * 2)], idx_v)

    iota2 = 2 * lax.iota(jnp.int32, _LANES)

    def step(k, carry):
        pos = k * (2 * _LANES) + iota2
        iv = plsc.load_gather(idx_v, [pos])
        jv = plsc.load_gather(idx_v, [pos + 1])
        ai = plsc.load_gather(a_v, [iv])
        bj = plsc.load_gather(b_v, [jv])
        aj = plsc.load_gather(a_v, [jv])
        bi = plsc.load_gather(b_v, [iv])
        out_v[pl.ds(k * _LANES, _LANES)] = ai * bj + aj * bi
        return carry

    lax.fori_loop(0, rows_per_w // _LANES, step, 0)

    pltpu.sync_copy(out_v, out_hbm.at[pl.ds(base, rows_per_w)])


def kernel(batch, A, B):
    n_rows = batch.shape[0]
    n_param = A.shape[0]

    info = plsc.get_sparse_core_info()
    num_cores, num_subcores = info.num_cores, info.num_subcores
    num_workers = num_cores * num_subcores
    rows_per_w = n_rows // num_workers

    mesh = plsc.VectorSubcoreMesh(core_axis_name="c", subcore_axis_name="s")
    batch_flat = batch.reshape(-1).astype(jnp.int32)

    k = functools.partial(
        pl.kernel,
        mesh=mesh,
        out_type=jax.ShapeDtypeStruct((n_rows,), jnp.float32),
        scratch_types=[
            pltpu.VMEM((rows_per_w * 2,), jnp.int32),
            pltpu.VMEM((n_param,), jnp.float32),
            pltpu.VMEM((n_param,), jnp.float32),
            pltpu.VMEM((rows_per_w,), jnp.float32),
        ],
    )(functools.partial(_demand_body, rows_per_w, num_cores))

    return k(batch_flat, A.astype(jnp.float32), B.astype(jnp.float32))


# speedup vs baseline: 31.0975x; 31.0975x over previous
import functools

import jax
import jax.numpy as jnp
from jax import lax
from jax.experimental import pallas as pl
from jax.experimental.pallas import tpu as pltpu
from jax.experimental.pallas import tpu_sc as plsc

_LANES = 16


def _demand_body(rows_per_w, num_cores, batch_hbm, a_hbm, b_hbm, out_hbm,
                 idx_v, a_v, b_v, out_v):
    c = lax.axis_index("c")
    s = lax.axis_index("s")
    wid = s * num_cores + c
    base = wid * rows_per_w

    pltpu.sync_copy(a_hbm, a_v)
    pltpu.sync_copy(b_hbm, b_v)
    pltpu.sync_copy(batch_hbm.at[pl.ds(base * 2, rows_per_w * 2)], idx_v)

    iota2 = 2 * lax.iota(jnp.int32, _LANES)

    def step(k, carry):
        pos = k * (2 * _LANES) + iota2
        iv = plsc.load_gather(idx_v, [pos])
        jv = plsc.load_gather(idx_v, [pos + 1])
        ai = plsc.load_gather(a_v, [iv])
        bj = plsc.load_gather(b_v, [jv])
        aj = plsc.load_gather(a_v, [jv])
        bi = plsc.load_gather(b_v, [iv])
        out_v[pl.ds(k * _LANES, _LANES)] = ai * bj + aj * bi
        return carry

    lax.fori_loop(0, rows_per_w // _LANES, step, 0)

    pltpu.sync_copy(out_v, out_hbm.at[pl.ds(base, rows_per_w)])


def kernel(batch, A, B):
    n_rows = batch.shape[0]
    n_param = A.shape[0]

    info = plsc.get_sparse_core_info()
    num_cores, num_subcores = info.num_cores, info.num_subcores
    num_workers = num_cores * num_subcores
    rows_per_w = n_rows // num_workers

    mesh = plsc.VectorSubcoreMesh(core_axis_name="c", subcore_axis_name="s")
    batch_flat = batch.reshape(-1).astype(jnp.int32)

    k = functools.partial(
        pl.kernel,
        mesh=mesh,
        out_type=jax.ShapeDtypeStruct((n_rows,), jnp.float32),
        scratch_types=[
            pltpu.VMEM((rows_per_w * 2,), jnp.int32),
            pltpu.VMEM((n_param,), jnp.float32),
            pltpu.VMEM((n_param,), jnp.float32),
            pltpu.VMEM((rows_per_w,), jnp.float32),
        ],
        compiler_params=pltpu.CompilerParams(needs_layout_passes=False),
    )(functools.partial(_demand_body, rows_per_w, num_cores))

    return k(batch_flat, A.astype(jnp.float32), B.astype(jnp.float32))


# speedup vs baseline: 31.5574x; 1.0148x over previous
import functools

import jax
import jax.numpy as jnp
from jax import lax
from jax.experimental import pallas as pl
from jax.experimental.pallas import tpu as pltpu
from jax.experimental.pallas import tpu_sc as plsc

_LANES = 16


def _demand_body(rows_per_w, num_cores, batch_hbm, a_hbm, b_hbm, out_hbm,
                 idx_v, a_v, b_v, out_v):
    c = lax.axis_index("c")
    s = lax.axis_index("s")
    wid = s * num_cores + c
    base = wid * rows_per_w

    pltpu.sync_copy(a_hbm, a_v)
    pltpu.sync_copy(b_hbm, b_v)
    pltpu.sync_copy(batch_hbm.at[pl.ds(base * 2, rows_per_w * 2)], idx_v)

    iota2 = 2 * lax.iota(jnp.int32, _LANES)

    @plsc.parallel_loop(0, rows_per_w, step=_LANES, unroll=8)
    def step(r):
        pos = r * 2 + iota2
        iv = plsc.load_gather(idx_v, [pos])
        jv = plsc.load_gather(idx_v, [pos + 1])
        ai = plsc.load_gather(a_v, [iv])
        bj = plsc.load_gather(b_v, [jv])
        aj = plsc.load_gather(a_v, [jv])
        bi = plsc.load_gather(b_v, [iv])
        out_v[pl.ds(r, _LANES)] = ai * bj + aj * bi

    pltpu.sync_copy(out_v, out_hbm.at[pl.ds(base, rows_per_w)])


def kernel(batch, A, B):
    n_rows = batch.shape[0]
    n_param = A.shape[0]

    info = plsc.get_sparse_core_info()
    num_cores, num_subcores = info.num_cores, info.num_subcores
    num_workers = num_cores * num_subcores
    rows_per_w = n_rows // num_workers

    mesh = plsc.VectorSubcoreMesh(core_axis_name="c", subcore_axis_name="s")
    batch_flat = batch.reshape(-1).astype(jnp.int32)

    k = functools.partial(
        pl.kernel,
        mesh=mesh,
        out_type=jax.ShapeDtypeStruct((n_rows,), jnp.float32),
        scratch_types=[
            pltpu.VMEM((rows_per_w * 2,), jnp.int32),
            pltpu.VMEM((n_param,), jnp.float32),
            pltpu.VMEM((n_param,), jnp.float32),
            pltpu.VMEM((rows_per_w,), jnp.float32),
        ],
        compiler_params=pltpu.CompilerParams(needs_layout_passes=False),
    )(functools.partial(_demand_body, rows_per_w, num_cores))

    return k(batch_flat, A.astype(jnp.float32), B.astype(jnp.float32))


# speedup vs baseline: 1227.3370x; 38.8923x over previous
import functools

import jax
import jax.numpy as jnp
from jax import lax
from jax.experimental import pallas as pl
from jax.experimental.pallas import tpu as pltpu
from jax.experimental.pallas import tpu_sc as plsc

_LANES = 16
_BLK = 128


def _demand_body(rows_per_w, num_cores, batch_hbm, a_hbm, b_hbm, out_hbm,
                 idx_v, a_v, b_v, out_v):
    c = lax.axis_index("c")
    s = lax.axis_index("s")
    wid = s * num_cores + c
    base = wid * rows_per_w

    pltpu.sync_copy(a_hbm, a_v)
    pltpu.sync_copy(b_hbm, b_v)
    pltpu.sync_copy(batch_hbm.at[pl.ds(base * 2, rows_per_w * 2)], idx_v)

    @plsc.parallel_loop(0, rows_per_w, step=_BLK, unroll=2)
    def step(r):
        w = r * 2
        for t in range(_BLK // _LANES):
            iv = idx_v[pl.ds(w + t * _LANES, _LANES)]
            jv = idx_v[pl.ds(w + _BLK + t * _LANES, _LANES)]
            ai = plsc.load_gather(a_v, [iv])
            bj = plsc.load_gather(b_v, [jv])
            aj = plsc.load_gather(a_v, [jv])
            bi = plsc.load_gather(b_v, [iv])
            out_v[pl.ds(r + t * _LANES, _LANES)] = ai * bj + aj * bi


def kernel(batch, A, B):
    n_rows = batch.shape[0]
    n_param = A.shape[0]

    info = plsc.get_sparse_core_info()
    num_cores, num_subcores = info.num_cores, info.num_subcores
    num_workers = num_cores * num_subcores
    rows_per_w = n_rows // num_workers

    batch_lin = jnp.transpose(
        batch.reshape(-1, _BLK, 2), (0, 2, 1)).reshape(-1)

    mesh = plsc.VectorSubcoreMesh(core_axis_name="c", subcore_axis_name="s")

    k = functools.partial(
        pl.kernel,
        mesh=mesh,
        out_type=jax.ShapeDtypeStruct((n_rows,), jnp.float32),
        scratch_types=[
            pltpu.VMEM((rows_per_w * 2,), jnp.int32),
            pltpu.VMEM((n_param,), jnp.float32),
            pltpu.VMEM((n_param,), jnp.float32),
            pltpu.VMEM((rows_per_w,), jnp.float32),
        ],
        compiler_params=pltpu.CompilerParams(needs_layout_passes=False),
    )(functools.partial(_demand_body, rows_per_w, num_cores))

    return k(batch_lin, A, B)
